# speedup vs baseline: 10.0589x; 10.0589x over previous
import functools

import jax
import jax.numpy as jnp
from jax import lax
from jax.experimental import pallas as pl
from jax.experimental.pallas import tpu as pltpu
from jax.experimental.pallas import tpu_sc as plsc

N = 10000
NP = 10240
E = 320000
EP = 327680
D = 128
NC = 2
NS = 16
NW = NC * NS
CHUNK = 128
ROWS_PER_TILE = EP // NW // CHUNK
PAD_NODE = 10200

_mesh = plsc.VectorSubcoreMesh(core_axis_name="c", subcore_axis_name="s")


@functools.partial(
    pl.kernel,
    out_type=[
        jax.ShapeDtypeStruct((NC, NP), jnp.float32),
        jax.ShapeDtypeStruct((NC, NP), jnp.float32),
    ],
    mesh=_mesh,
    scratch_types=[
        pltpu.VMEM_SHARED((NP,), jnp.float32),
        pltpu.VMEM_SHARED((NP,), jnp.float32),
        pltpu.VMEM((ROWS_PER_TILE, CHUNK), jnp.int32),
        pltpu.VMEM((ROWS_PER_TILE, CHUNK), jnp.int32),
        pltpu.VMEM((CHUNK,), jnp.float32),
        pltpu.VMEM((NP // NS,), jnp.float32),
    ],
)
def _degree_kernel(src_hbm, dst_hbm, dout_hbm, din_hbm,
                   acc_out, acc_in, idx_s, idx_d, ones_v, zer_v):
    c = lax.axis_index("c")
    s = lax.axis_index("s")
    for i in range(CHUNK // 16):
        ones_v[pl.ds(i * 16, 16)] = jnp.ones((16,), jnp.float32)
    for i in range((NP // NS) // 16):
        zer_v[pl.ds(i * 16, 16)] = jnp.zeros((16,), jnp.float32)
    off = s * (NP // NS)
    pltpu.sync_copy(zer_v, acc_out.at[pl.ds(off, NP // NS)])
    pltpu.sync_copy(zer_v, acc_in.at[pl.ds(off, NP // NS)])
    plsc.subcore_barrier()

    rb = (c * NS + s) * ROWS_PER_TILE
    pltpu.sync_copy(src_hbm.at[pl.ds(rb, ROWS_PER_TILE)], idx_s)
    pltpu.sync_copy(dst_hbm.at[pl.ds(rb, ROWS_PER_TILE)], idx_d)

    def body(j, carry):
        pltpu.sync_copy(ones_v, acc_out.at[idx_s.at[j]], add=True)
        pltpu.sync_copy(ones_v, acc_in.at[idx_d.at[j]], add=True)
        return carry

    lax.fori_loop(0, ROWS_PER_TILE, body, 0)
    plsc.subcore_barrier()
    pltpu.sync_copy(acc_out.at[pl.ds(off, NP // NS)],
                    dout_hbm.at[c, pl.ds(off, NP // NS)])
    pltpu.sync_copy(acc_in.at[pl.ds(off, NP // NS)],
                    din_hbm.at[c, pl.ds(off, NP // NS)])


@functools.partial(
    pl.kernel,
    out_type=jax.ShapeDtypeStruct((NC, NP, D), jnp.float32),
    mesh=_mesh,
    scratch_types=[
        pltpu.VMEM_SHARED((NP, D), jnp.float32),
        pltpu.VMEM((ROWS_PER_TILE, CHUNK), jnp.int32),
        pltpu.VMEM((ROWS_PER_TILE, CHUNK), jnp.int32),
        pltpu.VMEM((CHUNK, D), jnp.float32),
        pltpu.SemaphoreType.DMA,
    ],
)
def _agg_kernel(xs_hbm, src_hbm, dst_hbm, part_hbm,
                acc, idx_s, idx_d, rows, sem):
    c = lax.axis_index("c")
    s = lax.axis_index("s")

    def zbody(i, carry):
        for jj in range(D // 16):
            rows[i, pl.ds(jj * 16, 16)] = jnp.zeros((16,), jnp.float32)
        return carry

    lax.fori_loop(0, CHUNK, zbody, 0)
    roff = s * (NP // NS)
    for r in range((NP // NS) // CHUNK):
        pltpu.sync_copy(rows, acc.at[pl.ds(roff + r * CHUNK, CHUNK)])
    plsc.subcore_barrier()

    rb = (c * NS + s) * ROWS_PER_TILE
    pltpu.sync_copy(src_hbm.at[pl.ds(rb, ROWS_PER_TILE)], idx_s)
    pltpu.sync_copy(dst_hbm.at[pl.ds(rb, ROWS_PER_TILE)], idx_d)

    def body(j, carry):
        pltpu.async_copy(xs_hbm.at[idx_s.at[j]], rows, sem).wait()
        pltpu.sync_copy(rows, acc.at[idx_d.at[j]], add=True)
        return carry

    lax.fori_loop(0, ROWS_PER_TILE, body, 0)
    plsc.subcore_barrier()
    pltpu.sync_copy(acc.at[pl.ds(roff, NP // NS)],
                    part_hbm.at[c, pl.ds(roff, NP // NS)])


def _scale_body(x_ref, dp_ref, o_ref):
    d = dp_ref[0] + dp_ref[1]
    o_ref[...] = x_ref[...] * lax.rsqrt(jnp.maximum(d, 1.0))


def _scale_call(xp, dout_p):
    R = 2048
    grid = (NP // R,)
    return pl.pallas_call(
        _scale_body,
        grid=grid,
        in_specs=[
            pl.BlockSpec((R, D), lambda i: (i, 0)),
            pl.BlockSpec((2, R, 1), lambda i: (0, i, 0)),
        ],
        out_specs=pl.BlockSpec((R, D), lambda i: (i, 0)),
        out_shape=jax.ShapeDtypeStruct((NP, D), jnp.float32),
    )(xp, dout_p)


def _combine_body(p_ref, dp_ref, w_ref, o_ref):
    d = dp_ref[0] + dp_ref[1]
    agg = (p_ref[0] + p_ref[1]) * lax.rsqrt(jnp.maximum(d, 1.0))
    o_ref[...] = jnp.dot(agg, w_ref[...], preferred_element_type=jnp.float32)


def _combine_call(part, din_p, W):
    R = 2000
    grid = (N // R,)
    return pl.pallas_call(
        _combine_body,
        grid=grid,
        in_specs=[
            pl.BlockSpec((2, R, D), lambda i: (0, i, 0)),
            pl.BlockSpec((2, R, 1), lambda i: (0, i, 0)),
            pl.BlockSpec((D, D), lambda i: (0, 0)),
        ],
        out_specs=pl.BlockSpec((R, D), lambda i: (i, 0)),
        out_shape=jax.ShapeDtypeStruct((N, D), jnp.float32),
    )(part, din_p, W)


def kernel(x, edge_index, W):
    x = x.astype(jnp.float32)
    ei = edge_index.astype(jnp.int32)
    pad = jnp.full((EP - E,), PAD_NODE, jnp.int32)
    src2d = jnp.concatenate([ei[0], pad]).reshape(EP // CHUNK, CHUNK)
    dst2d = jnp.concatenate([ei[1], pad]).reshape(EP // CHUNK, CHUNK)

    dout_p, din_p = _degree_kernel(src2d, dst2d)

    xp = jnp.zeros((NP, D), jnp.float32).at[:N].set(x)
    xs = _scale_call(xp, dout_p.reshape(NC, NP, 1))

    part = _agg_kernel(xs, src2d, dst2d)

    out = _combine_call(part[:, :N], din_p.reshape(NC, NP, 1)[:, :N], W)
    return out


# speedup vs baseline: 10.7705x; 1.0707x over previous
import functools

import jax
import jax.numpy as jnp
from jax import lax
from jax.experimental import pallas as pl
from jax.experimental.pallas import tpu as pltpu
from jax.experimental.pallas import tpu_sc as plsc

N = 10000
NP = 10240
E = 320000
EP = 327680
D = 128
NC = 2
NS = 16
CHUNK = 128
NCH = EP // (NC * NS) // CHUNK
PAD_NODE = 10200
NIB = 4

_mesh = plsc.VectorSubcoreMesh(core_axis_name="c", subcore_axis_name="s")


@functools.partial(
    pl.kernel,
    out_type=[
        jax.ShapeDtypeStruct((NC, NP), jnp.float32),
        jax.ShapeDtypeStruct((NC, NP), jnp.float32),
    ],
    mesh=_mesh,
    scratch_types=[
        pltpu.VMEM_SHARED((NP,), jnp.float32),
        pltpu.VMEM_SHARED((NP,), jnp.float32),
        pltpu.VMEM((NCH, CHUNK), jnp.int32),
        pltpu.VMEM((NCH, CHUNK), jnp.int32),
        pltpu.VMEM((CHUNK,), jnp.float32),
        pltpu.VMEM((NP // NS,), jnp.float32),
    ],
)
def _degree_kernel(src_hbm, dst_hbm, dout_hbm, din_hbm,
                   acc_out, acc_in, idx_s, idx_d, ones_v, zer_v):
    c = lax.axis_index("c")
    s = lax.axis_index("s")
    for i in range(CHUNK // 16):
        ones_v[pl.ds(i * 16, 16)] = jnp.ones((16,), jnp.float32)
    for i in range((NP // NS) // 16):
        zer_v[pl.ds(i * 16, 16)] = jnp.zeros((16,), jnp.float32)
    off = s * (NP // NS)
    pltpu.sync_copy(zer_v, acc_out.at[pl.ds(off, NP // NS)])
    pltpu.sync_copy(zer_v, acc_in.at[pl.ds(off, NP // NS)])
    plsc.subcore_barrier()

    rb = (c * NS + s) * NCH
    pltpu.sync_copy(src_hbm.at[pl.ds(rb, NCH)], idx_s)
    pltpu.sync_copy(dst_hbm.at[pl.ds(rb, NCH)], idx_d)

    def body(j, carry):
        pltpu.sync_copy(ones_v, acc_out.at[idx_s.at[j]], add=True)
        pltpu.sync_copy(ones_v, acc_in.at[idx_d.at[j]], add=True)
        return carry

    lax.fori_loop(0, NCH, body, 0)
    plsc.subcore_barrier()
    pltpu.sync_copy(acc_out.at[pl.ds(off, NP // NS)],
                    dout_hbm.at[c, pl.ds(off, NP // NS)])
    pltpu.sync_copy(acc_in.at[pl.ds(off, NP // NS)],
                    din_hbm.at[c, pl.ds(off, NP // NS)])


@functools.partial(
    pl.kernel,
    out_type=jax.ShapeDtypeStruct((NC, NP, D), jnp.float32),
    mesh=_mesh,
    scratch_types=[
        pltpu.VMEM_SHARED((NP, D), jnp.float32),
        pltpu.VMEM((NIB, CHUNK), jnp.int32),
        pltpu.VMEM((NIB, CHUNK), jnp.int32),
        pltpu.VMEM((2, CHUNK, D), jnp.float32),
        [pltpu.SemaphoreType.DMA] * 2,
        [pltpu.SemaphoreType.DMA] * 2,
        [pltpu.SemaphoreType.DMA] * NIB,
    ],
)
def _agg_kernel(xs_hbm, src_hbm, dst_hbm, part_hbm,
                acc, idx_s, idx_d, rows, sg, ss, si):
    c = lax.axis_index("c")
    s = lax.axis_index("s")

    def zbody(i, carry):
        for jj in range(D // 16):
            rows[0, i, pl.ds(jj * 16, 16)] = jnp.zeros((16,), jnp.float32)
        return carry

    lax.fori_loop(0, CHUNK, zbody, 0)
    roff = s * (NP // NS)
    for r in range((NP // NS) // CHUNK):
        pltpu.sync_copy(rows.at[0], acc.at[pl.ds(roff + r * CHUNK, CHUNK)])
    plsc.subcore_barrier()

    rb = (c * NS + s) * NCH

    def i_issue(j, q):
        pltpu.async_copy(src_hbm.at[rb + j], idx_s.at[q], si[q])
        pltpu.async_copy(dst_hbm.at[rb + j], idx_d.at[q], si[q])

    def i_wait(j, q):
        pltpu.make_async_copy(src_hbm.at[rb + j], idx_s.at[q], si[q]).wait()
        pltpu.make_async_copy(dst_hbm.at[rb + j], idx_d.at[q], si[q]).wait()

    def g_issue(q, r):
        pltpu.async_copy(xs_hbm.at[idx_s.at[q]], rows.at[r], sg[r])

    def g_wait(q, r):
        pltpu.make_async_copy(xs_hbm.at[idx_s.at[q]], rows.at[r], sg[r]).wait()

    def s_issue(q, r):
        pltpu.async_copy(rows.at[r], acc.at[idx_d.at[q]], ss[r], add=True)

    def s_wait(q, r):
        pltpu.make_async_copy(rows.at[r], acc.at[idx_d.at[q]], ss[r]).wait()

    def group(j0, first):
        g_wait(0, 0)
        if not first:
            s_wait(3, 1)
        i_issue(j0 + 3, 3)
        i_wait(j0 + 1, 1)
        g_issue(1, 1)
        s_issue(0, 0)
        g_wait(1, 1)
        s_wait(0, 0)
        i_issue(j0 + 4, 0)
        i_wait(j0 + 2, 2)
        g_issue(2, 0)
        s_issue(1, 1)
        g_wait(2, 0)
        s_wait(1, 1)
        i_issue(j0 + 5, 1)
        i_wait(j0 + 3, 3)
        g_issue(3, 1)
        s_issue(2, 0)
        g_wait(3, 1)
        s_wait(2, 0)
        i_issue(j0 + 6, 2)
        i_wait(j0 + 4, 0)
        g_issue(0, 0)
        s_issue(3, 1)

    i_issue(0, 0)
    i_issue(1, 1)
    i_issue(2, 2)
    i_wait(0, 0)
    g_issue(0, 0)
    group(0, first=True)

    def body(h, carry):
        group(4 * h, first=False)
        return carry

    lax.fori_loop(1, NCH // 4 - 1, body, 0)

    j0 = NCH - 4
    g_wait(0, 0)
    s_wait(3, 1)
    i_issue(j0 + 3, 3)
    i_wait(j0 + 1, 1)
    g_issue(1, 1)
    s_issue(0, 0)
    g_wait(1, 1)
    s_wait(0, 0)
    i_wait(j0 + 2, 2)
    g_issue(2, 0)
    s_issue(1, 1)
    g_wait(2, 0)
    s_wait(1, 1)
    i_wait(j0 + 3, 3)
    g_issue(3, 1)
    s_issue(2, 0)
    g_wait(3, 1)
    s_wait(2, 0)
    s_issue(3, 1)
    s_wait(3, 1)

    plsc.subcore_barrier()
    pltpu.sync_copy(acc.at[pl.ds(roff, NP // NS)],
                    part_hbm.at[c, pl.ds(roff, NP // NS)])


def _scale_body(x_ref, dp_ref, o_ref):
    d = dp_ref[0] + dp_ref[1]
    o_ref[...] = x_ref[...] * lax.rsqrt(jnp.maximum(d, 1.0))


def _scale_call(xp, dout_p):
    R = 2048
    grid = (NP // R,)
    return pl.pallas_call(
        _scale_body,
        grid=grid,
        in_specs=[
            pl.BlockSpec((R, D), lambda i: (i, 0)),
            pl.BlockSpec((2, R, 1), lambda i: (0, i, 0)),
        ],
        out_specs=pl.BlockSpec((R, D), lambda i: (i, 0)),
        out_shape=jax.ShapeDtypeStruct((NP, D), jnp.float32),
    )(xp, dout_p)


def _combine_body(p_ref, dp_ref, w_ref, o_ref):
    d = dp_ref[0] + dp_ref[1]
    agg = (p_ref[0] + p_ref[1]) * lax.rsqrt(jnp.maximum(d, 1.0))
    o_ref[...] = jnp.dot(agg, w_ref[...], preferred_element_type=jnp.float32)


def _combine_call(part, din_p, W):
    R = 2000
    grid = (N // R,)
    return pl.pallas_call(
        _combine_body,
        grid=grid,
        in_specs=[
            pl.BlockSpec((2, R, D), lambda i: (0, i, 0)),
            pl.BlockSpec((2, R, 1), lambda i: (0, i, 0)),
            pl.BlockSpec((D, D), lambda i: (0, 0)),
        ],
        out_specs=pl.BlockSpec((R, D), lambda i: (i, 0)),
        out_shape=jax.ShapeDtypeStruct((N, D), jnp.float32),
    )(part, din_p, W)


def kernel(x, edge_index, W):
    x = x.astype(jnp.float32)
    ei = edge_index.astype(jnp.int32)
    pad = jnp.full((EP - E,), PAD_NODE, jnp.int32)
    src2d = jnp.concatenate([ei[0], pad]).reshape(EP // CHUNK, CHUNK)
    dst2d = jnp.concatenate([ei[1], pad]).reshape(EP // CHUNK, CHUNK)

    dout_p, din_p = _degree_kernel(src2d, dst2d)

    xp = jnp.zeros((NP, D), jnp.float32).at[:N].set(x)
    xs = _scale_call(xp, dout_p.reshape(NC, NP, 1))

    part = _agg_kernel(xs, src2d, dst2d)

    out = _combine_call(part[:, :N], din_p.reshape(NC, NP, 1)[:, :N], W)
    return out


# speedup vs baseline: 28.6153x; 2.6568x over previous
import functools

import jax
import jax.numpy as jnp
from jax import lax
from jax.experimental import pallas as pl
from jax.experimental.pallas import tpu as pltpu
from jax.experimental.pallas import tpu_sc as plsc

N = 10000
NP = 10240
E = 320000
EP = 327680
D = 128
DH = D // 2
NC = 2
NS = 16
CHUNK = 128
NCHD = EP // (NC * NS) // CHUNK
NCH = EP // NS // CHUNK
PAD_NODE = 10200
NIB = 4

_mesh = plsc.VectorSubcoreMesh(core_axis_name="c", subcore_axis_name="s")


@functools.partial(
    pl.kernel,
    out_type=[
        jax.ShapeDtypeStruct((NC, NP), jnp.float32),
        jax.ShapeDtypeStruct((NC, NP), jnp.float32),
    ],
    mesh=_mesh,
    scratch_types=[
        pltpu.VMEM_SHARED((NP,), jnp.float32),
        pltpu.VMEM_SHARED((NP,), jnp.float32),
        pltpu.VMEM((NCHD, CHUNK), jnp.int32),
        pltpu.VMEM((NCHD, CHUNK), jnp.int32),
        pltpu.VMEM((CHUNK,), jnp.float32),
        pltpu.VMEM((NP // NS,), jnp.float32),
    ],
)
def _degree_kernel(src_hbm, dst_hbm, dout_hbm, din_hbm,
                   acc_out, acc_in, idx_s, idx_d, ones_v, zer_v):
    c = lax.axis_index("c")
    s = lax.axis_index("s")
    for i in range(CHUNK // 16):
        ones_v[pl.ds(i * 16, 16)] = jnp.ones((16,), jnp.float32)
    for i in range((NP // NS) // 16):
        zer_v[pl.ds(i * 16, 16)] = jnp.zeros((16,), jnp.float32)
    off = s * (NP // NS)
    pltpu.sync_copy(zer_v, acc_out.at[pl.ds(off, NP // NS)])
    pltpu.sync_copy(zer_v, acc_in.at[pl.ds(off, NP // NS)])
    plsc.subcore_barrier()

    rb = (c * NS + s) * NCHD
    pltpu.sync_copy(src_hbm.at[pl.ds(rb, NCHD)], idx_s)
    pltpu.sync_copy(dst_hbm.at[pl.ds(rb, NCHD)], idx_d)

    def body(j, carry):
        pltpu.sync_copy(ones_v, acc_out.at[idx_s.at[j]], add=True)
        pltpu.sync_copy(ones_v, acc_in.at[idx_d.at[j]], add=True)
        return carry

    lax.fori_loop(0, NCHD, body, 0)
    plsc.subcore_barrier()
    pltpu.sync_copy(acc_out.at[pl.ds(off, NP // NS)],
                    dout_hbm.at[c, pl.ds(off, NP // NS)])
    pltpu.sync_copy(acc_in.at[pl.ds(off, NP // NS)],
                    din_hbm.at[c, pl.ds(off, NP // NS)])


@functools.partial(
    pl.kernel,
    out_type=jax.ShapeDtypeStruct((NC, NP, DH), jnp.float32),
    mesh=_mesh,
    scratch_types=[
        pltpu.VMEM_SHARED((NP, DH), jnp.float32),
        pltpu.VMEM_SHARED((NP, DH), jnp.float32),
        pltpu.VMEM((NIB, CHUNK), jnp.int32),
        pltpu.VMEM((NIB, CHUNK), jnp.int32),
        pltpu.VMEM((2, CHUNK, DH), jnp.float32),
        [pltpu.SemaphoreType.DMA] * 2,
        [pltpu.SemaphoreType.DMA] * 2,
        [pltpu.SemaphoreType.DMA] * NIB,
    ],
)
def _agg_kernel(xs_hbm, src_hbm, dst_hbm, part_hbm,
                xsl, acc, idx_s, idx_d, rows, sg, ss, si):
    c = lax.axis_index("c")
    s = lax.axis_index("s")

    roff = s * (NP // NS)
    pltpu.sync_copy(xs_hbm.at[c, pl.ds(roff, NP // NS)],
                    xsl.at[pl.ds(roff, NP // NS)])

    def zbody(i, carry):
        for jj in range(DH // 16):
            rows[0, i, pl.ds(jj * 16, 16)] = jnp.zeros((16,), jnp.float32)
        return carry

    lax.fori_loop(0, CHUNK, zbody, 0)
    for r in range((NP // NS) // CHUNK):
        pltpu.sync_copy(rows.at[0], acc.at[pl.ds(roff + r * CHUNK, CHUNK)])
    plsc.subcore_barrier()

    rb = s * NCH

    def i_issue(j, q):
        pltpu.async_copy(src_hbm.at[rb + j], idx_s.at[q], si[q])
        pltpu.async_copy(dst_hbm.at[rb + j], idx_d.at[q], si[q])

    def i_wait(j, q):
        pltpu.make_async_copy(src_hbm.at[rb + j], idx_s.at[q], si[q]).wait()
        pltpu.make_async_copy(dst_hbm.at[rb + j], idx_d.at[q], si[q]).wait()

    def g_issue(q, r):
        pltpu.async_copy(xsl.at[idx_s.at[q]], rows.at[r], sg[r])

    def g_wait(q, r):
        pltpu.make_async_copy(xsl.at[idx_s.at[q]], rows.at[r], sg[r]).wait()

    def s_issue(q, r):
        pltpu.async_copy(rows.at[r], acc.at[idx_d.at[q]], ss[r], add=True)

    def s_wait(q, r):
        pltpu.make_async_copy(rows.at[r], acc.at[idx_d.at[q]], ss[r]).wait()

    def group(j0, first):
        g_wait(0, 0)
        if not first:
            s_wait(3, 1)
        i_issue(j0 + 3, 3)
        i_wait(j0 + 1, 1)
        g_issue(1, 1)
        s_issue(0, 0)
        g_wait(1, 1)
        s_wait(0, 0)
        i_issue(j0 + 4, 0)
        i_wait(j0 + 2, 2)
        g_issue(2, 0)
        s_issue(1, 1)
        g_wait(2, 0)
        s_wait(1, 1)
        i_issue(j0 + 5, 1)
        i_wait(j0 + 3, 3)
        g_issue(3, 1)
        s_issue(2, 0)
        g_wait(3, 1)
        s_wait(2, 0)
        i_issue(j0 + 6, 2)
        i_wait(j0 + 4, 0)
        g_issue(0, 0)
        s_issue(3, 1)

    i_issue(0, 0)
    i_issue(1, 1)
    i_issue(2, 2)
    i_wait(0, 0)
    g_issue(0, 0)
    group(0, first=True)

    def body(h, carry):
        group(4 * h, first=False)
        return carry

    lax.fori_loop(1, NCH // 4 - 1, body, 0)

    j0 = NCH - 4
    g_wait(0, 0)
    s_wait(3, 1)
    i_issue(j0 + 3, 3)
    i_wait(j0 + 1, 1)
    g_issue(1, 1)
    s_issue(0, 0)
    g_wait(1, 1)
    s_wait(0, 0)
    i_wait(j0 + 2, 2)
    g_issue(2, 0)
    s_issue(1, 1)
    g_wait(2, 0)
    s_wait(1, 1)
    i_wait(j0 + 3, 3)
    g_issue(3, 1)
    s_issue(2, 0)
    g_wait(3, 1)
    s_wait(2, 0)
    s_issue(3, 1)
    s_wait(3, 1)

    plsc.subcore_barrier()
    pltpu.sync_copy(acc.at[pl.ds(roff, NP // NS)],
                    part_hbm.at[c, pl.ds(roff, NP // NS)])


def _scale_body(x_ref, dp_ref, o_ref):
    d = dp_ref[0] + dp_ref[1]
    xs = x_ref[...] * lax.rsqrt(jnp.maximum(d, 1.0))
    o_ref[0] = xs[:, :DH]
    o_ref[1] = xs[:, DH:]


def _scale_call(xp, dout_p):
    R = 2048
    grid = (NP // R,)
    return pl.pallas_call(
        _scale_body,
        grid=grid,
        in_specs=[
            pl.BlockSpec((R, D), lambda i: (i, 0)),
            pl.BlockSpec((2, R, 1), lambda i: (0, i, 0)),
        ],
        out_specs=pl.BlockSpec((2, R, DH), lambda i: (0, i, 0)),
        out_shape=jax.ShapeDtypeStruct((NC, NP, DH), jnp.float32),
    )(xp, dout_p)


def _combine_body(p_ref, dp_ref, w_ref, o_ref):
    d = dp_ref[0] + dp_ref[1]
    agg = jnp.concatenate([p_ref[0], p_ref[1]], axis=1)
    agg = agg * lax.rsqrt(jnp.maximum(d, 1.0))
    o_ref[...] = jnp.dot(agg, w_ref[...], preferred_element_type=jnp.float32)


def _combine_call(part, din_p, W):
    R = 2000
    grid = (N // R,)
    return pl.pallas_call(
        _combine_body,
        grid=grid,
        in_specs=[
            pl.BlockSpec((2, R, DH), lambda i: (0, i, 0)),
            pl.BlockSpec((2, R, 1), lambda i: (0, i, 0)),
            pl.BlockSpec((D, D), lambda i: (0, 0)),
        ],
        out_specs=pl.BlockSpec((R, D), lambda i: (i, 0)),
        out_shape=jax.ShapeDtypeStruct((N, D), jnp.float32),
    )(part, din_p, W)


def kernel(x, edge_index, W):
    x = x.astype(jnp.float32)
    ei = edge_index.astype(jnp.int32)
    pad = jnp.full((EP - E,), PAD_NODE, jnp.int32)
    src2d = jnp.concatenate([ei[0], pad]).reshape(EP // CHUNK, CHUNK)
    dst2d = jnp.concatenate([ei[1], pad]).reshape(EP // CHUNK, CHUNK)

    dout_p, din_p = _degree_kernel(src2d, dst2d)

    xp = jnp.zeros((NP, D), jnp.float32).at[:N].set(x)
    xs = _scale_call(xp, dout_p.reshape(NC, NP, 1))

    part = _agg_kernel(xs, src2d, dst2d)

    out = _combine_call(part[:, :N], din_p.reshape(NC, NP, 1)[:, :N], W)
    return out


# speedup vs baseline: 30.0431x; 1.0499x over previous
import functools

import jax
import jax.numpy as jnp
from jax import lax
from jax.experimental import pallas as pl
from jax.experimental.pallas import tpu as pltpu
from jax.experimental.pallas import tpu_sc as plsc

N = 10000
NP = 10240
E = 320000
EP = 327680
D = 128
DH = D // 2
NC = 2
NS = 16
CHUNK = 128
NCHD = EP // (NC * NS) // CHUNK
NCH = EP // NS // CHUNK
PAD_NODE = 10200
RB = 4
NIB = 8

_mesh = plsc.VectorSubcoreMesh(core_axis_name="c", subcore_axis_name="s")


@functools.partial(
    pl.kernel,
    out_type=[
        jax.ShapeDtypeStruct((NC, NP), jnp.float32),
        jax.ShapeDtypeStruct((NC, NP), jnp.float32),
    ],
    mesh=_mesh,
    scratch_types=[
        pltpu.VMEM_SHARED((NP,), jnp.float32),
        pltpu.VMEM_SHARED((NP,), jnp.float32),
        pltpu.VMEM((NCHD, CHUNK), jnp.int32),
        pltpu.VMEM((NCHD, CHUNK), jnp.int32),
        pltpu.VMEM((CHUNK,), jnp.float32),
        pltpu.VMEM((NP // NS,), jnp.float32),
        [pltpu.SemaphoreType.DMA] * 4,
        [pltpu.SemaphoreType.DMA] * 4,
    ],
)
def _degree_kernel(src_hbm, dst_hbm, dout_hbm, din_hbm,
                   acc_out, acc_in, idx_s, idx_d, ones_v, zer_v, so, sd):
    c = lax.axis_index("c")
    s = lax.axis_index("s")
    for i in range(CHUNK // 16):
        ones_v[pl.ds(i * 16, 16)] = jnp.ones((16,), jnp.float32)
    for i in range((NP // NS) // 16):
        zer_v[pl.ds(i * 16, 16)] = jnp.zeros((16,), jnp.float32)
    off = s * (NP // NS)
    pltpu.sync_copy(zer_v, acc_out.at[pl.ds(off, NP // NS)])
    pltpu.sync_copy(zer_v, acc_in.at[pl.ds(off, NP // NS)])
    plsc.subcore_barrier()

    rb = (c * NS + s) * NCHD
    pltpu.sync_copy(src_hbm.at[pl.ds(rb, NCHD)], idx_s)
    pltpu.sync_copy(dst_hbm.at[pl.ds(rb, NCHD)], idx_d)

    def body(h, carry):
        j0 = 4 * h
        for k in range(4):
            pltpu.async_copy(ones_v, acc_out.at[idx_s.at[j0 + k]], so[k], add=True)
            pltpu.async_copy(ones_v, acc_in.at[idx_d.at[j0 + k]], sd[k], add=True)
        for k in range(4):
            pltpu.make_async_copy(ones_v, acc_out.at[idx_s.at[j0 + k]], so[k]).wait()
            pltpu.make_async_copy(ones_v, acc_in.at[idx_d.at[j0 + k]], sd[k]).wait()
        return carry

    lax.fori_loop(0, NCHD // 4, body, 0)
    plsc.subcore_barrier()
    pltpu.sync_copy(acc_out.at[pl.ds(off, NP // NS)],
                    dout_hbm.at[c, pl.ds(off, NP // NS)])
    pltpu.sync_copy(acc_in.at[pl.ds(off, NP // NS)],
                    din_hbm.at[c, pl.ds(off, NP // NS)])


@functools.partial(
    pl.kernel,
    out_type=jax.ShapeDtypeStruct((NC, NP, DH), jnp.float32),
    mesh=_mesh,
    scratch_types=[
        pltpu.VMEM_SHARED((NP, DH), jnp.float32),
        pltpu.VMEM_SHARED((NP, DH), jnp.float32),
        pltpu.VMEM((NIB, CHUNK), jnp.int32),
        pltpu.VMEM((NIB, CHUNK), jnp.int32),
        pltpu.VMEM((2, CHUNK, DH), jnp.float32),
        [pltpu.SemaphoreType.DMA] * 2,
        [pltpu.SemaphoreType.DMA] * 2,
        [pltpu.SemaphoreType.DMA] * NIB,
    ],
)
def _agg_kernel(xs_hbm, src_hbm, dst_hbm, part_hbm,
                xsl, acc, idx_s, idx_d, rows, sg, ss, si):
    c = lax.axis_index("c")
    s = lax.axis_index("s")

    roff = s * (NP // NS)
    pltpu.sync_copy(xs_hbm.at[c, pl.ds(roff, NP // NS)],
                    xsl.at[pl.ds(roff, NP // NS)])

    def zbody(i, carry):
        for jj in range(DH // 16):
            rows[0, i, pl.ds(jj * 16, 16)] = jnp.zeros((16,), jnp.float32)
        return carry

    lax.fori_loop(0, CHUNK, zbody, 0)
    for r in range((NP // NS) // CHUNK):
        pltpu.sync_copy(rows.at[0], acc.at[pl.ds(roff + r * CHUNK, CHUNK)])
    plsc.subcore_barrier()

    rbase = s * NCH

    def i_issue(j, q):
        pltpu.async_copy(src_hbm.at[rbase + j], idx_s.at[q], si[q])
        pltpu.async_copy(dst_hbm.at[rbase + j], idx_d.at[q], si[q])

    def i_wait(j, q):
        pltpu.make_async_copy(src_hbm.at[rbase + j], idx_s.at[q], si[q]).wait()
        pltpu.make_async_copy(dst_hbm.at[rbase + j], idx_d.at[q], si[q]).wait()

    def g_issue(q, r):
        pltpu.async_copy(xsl.at[idx_s.at[q]], rows.at[r], sg[r])

    def g_wait(q, r):
        pltpu.make_async_copy(xsl.at[idx_s.at[q]], rows.at[r], sg[r]).wait()

    def s_issue(q, r):
        pltpu.async_copy(rows.at[r], acc.at[idx_d.at[q]], ss[r], add=True)

    def s_wait(q, r):
        pltpu.make_async_copy(rows.at[r], acc.at[idx_d.at[q]], ss[r]).wait()

    def group(j0, first):
        g_wait(0, 0)
        if not first:
            s_wait(3, 1)
        i_issue(j0 + 3, 3)
        i_wait(j0 + 1, 1)
        g_issue(1, 1)
        s_issue(0, 0)
        g_wait(1, 1)
        s_wait(0, 0)
        i_issue(j0 + 4, 0)
        i_wait(j0 + 2, 2)
        g_issue(2, 0)
        s_issue(1, 1)
        g_wait(2, 0)
        s_wait(1, 1)
        i_issue(j0 + 5, 1)
        i_wait(j0 + 3, 3)
        g_issue(3, 1)
        s_issue(2, 0)
        g_wait(3, 1)
        s_wait(2, 0)
        i_issue(j0 + 6, 2)
        i_wait(j0 + 4, 0)
        g_issue(0, 0)
        s_issue(3, 1)

    i_issue(0, 0)
    i_issue(1, 1)
    i_issue(2, 2)
    i_wait(0, 0)
    g_issue(0, 0)
    group(0, first=True)

    def body(h, carry):
        group(4 * h, first=False)
        return carry

    lax.fori_loop(1, NCH // 4 - 1, body, 0)

    j0 = NCH - 4
    g_wait(0, 0)
    s_wait(3, 1)
    i_issue(j0 + 3, 3)
    i_wait(j0 + 1, 1)
    g_issue(1, 1)
    s_issue(0, 0)
    g_wait(1, 1)
    s_wait(0, 0)
    i_wait(j0 + 2, 2)
    g_issue(2, 0)
    s_issue(1, 1)
    g_wait(2, 0)
    s_wait(1, 1)
    i_wait(j0 + 3, 3)
    g_issue(3, 1)
    s_issue(2, 0)
    g_wait(3, 1)
    s_wait(2, 0)
    s_issue(3, 1)
    s_wait(3, 1)

    plsc.subcore_barrier()
    pltpu.sync_copy(acc.at[pl.ds(roff, NP // NS)],
                    part_hbm.at[c, pl.ds(roff, NP // NS)])


def _scale_body(x_ref, dp_ref, o_ref):
    d = dp_ref[0] + dp_ref[1]
    xs = x_ref[...] * lax.rsqrt(jnp.maximum(d, 1.0))
    o_ref[0] = xs[:, :DH]
    o_ref[1] = xs[:, DH:]


def _scale_call(xp, dout_p):
    R = 2048
    grid = (NP // R,)
    return pl.pallas_call(
        _scale_body,
        grid=grid,
        in_specs=[
            pl.BlockSpec((R, D), lambda i: (i, 0)),
            pl.BlockSpec((2, R, 1), lambda i: (0, i, 0)),
        ],
        out_specs=pl.BlockSpec((2, R, DH), lambda i: (0, i, 0)),
        out_shape=jax.ShapeDtypeStruct((NC, NP, DH), jnp.float32),
    )(xp, dout_p)


def _combine_body(p_ref, dp_ref, w_ref, o_ref):
    d = dp_ref[0] + dp_ref[1]
    agg = jnp.concatenate([p_ref[0], p_ref[1]], axis=1)
    agg = agg * lax.rsqrt(jnp.maximum(d, 1.0))
    o_ref[...] = jnp.dot(agg, w_ref[...], preferred_element_type=jnp.float32)


def _combine_call(part, din_p, W):
    R = 2000
    grid = (N // R,)
    return pl.pallas_call(
        _combine_body,
        grid=grid,
        in_specs=[
            pl.BlockSpec((2, R, DH), lambda i: (0, i, 0)),
            pl.BlockSpec((2, R, 1), lambda i: (0, i, 0)),
            pl.BlockSpec((D, D), lambda i: (0, 0)),
        ],
        out_specs=pl.BlockSpec((R, D), lambda i: (i, 0)),
        out_shape=jax.ShapeDtypeStruct((N, D), jnp.float32),
    )(part, din_p, W)


def kernel(x, edge_index, W):
    x = x.astype(jnp.float32)
    ei = edge_index.astype(jnp.int32)
    pad = jnp.full((EP - E,), PAD_NODE, jnp.int32)
    src2d = jnp.concatenate([ei[0], pad]).reshape(EP // CHUNK, CHUNK)
    dst2d = jnp.concatenate([ei[1], pad]).reshape(EP // CHUNK, CHUNK)

    dout_p, din_p = _degree_kernel(src2d, dst2d)

    xp = jnp.zeros((NP, D), jnp.float32).at[:N].set(x)
    xs = _scale_call(xp, dout_p.reshape(NC, NP, 1))

    part = _agg_kernel(xs, src2d, dst2d)

    out = _combine_call(part, din_p.reshape(NC, NP, 1), W)
    return out
